# baseline (speedup 1.0000x reference)
import jax
import jax.numpy as jnp
from jax import lax
from jax.experimental import pallas as pl
from jax.experimental.pallas import tpu as pltpu
from jax.experimental.pallas import tpu_sc as plsc

ROWS = 4096
D_IN = 4096

NC = 2
NS = 16
BLK = 128
NBLK_PER_SC = D_IN // NC // BLK
BLK_FLOATS = BLK * ROWS
TILE_SLICE = BLK_FLOATS // NS
ZCHUNK = 16384
ROWLEN = 128


def _build_scatter(ch_per_tile):
    nrows = ch_per_tile // ROWLEN

    def body(child_hbm, parent_hbm, val_hbm, wt_hbm,
             child_v, parent_v, val_v, fidx_v, mval_v, zeros_v, flush_v,
             shared_v):
        c = lax.axis_index("c")
        s = lax.axis_index("s")
        base = s * ch_per_tile
        pltpu.sync_copy(child_hbm.at[pl.ds(base, ch_per_tile)], child_v)
        pltpu.sync_copy(parent_hbm.at[pl.ds(base, ch_per_tile)], parent_v)
        pltpu.sync_copy(val_hbm.at[pl.ds(base, ch_per_tile)], val_v)

        def zinit(i, carry):
            zeros_v[pl.ds(i * 16, 16)] = jnp.zeros((16,), jnp.float32)
            return carry
        lax.fori_loop(0, ZCHUNK // 16, zinit, 0)

        for b in range(NBLK_PER_SC):
            gblk = c * NBLK_PER_SC + b
            p0 = gblk * BLK
            for z in range(TILE_SLICE // ZCHUNK):
                pltpu.sync_copy(
                    zeros_v,
                    shared_v.at[pl.ds(s * TILE_SLICE + z * ZCHUNK, ZCHUNK)])
            plsc.subcore_barrier()

            def row_loop(j, carry):
                for u in range(ROWLEN // 16):
                    o = j * ROWLEN + u * 16
                    ch = child_v[pl.ds(o, 16)]
                    pa = parent_v[pl.ds(o, 16)]
                    va = val_v[pl.ds(o, 16)]
                    rel = pa - p0
                    inb = (rel >= 0) & (rel < BLK)
                    fidx_v[j, pl.ds(u * 16, 16)] = jnp.where(
                        inb, rel * ROWS + ch, 0)
                    mval_v[j, pl.ds(u * 16, 16)] = jnp.where(inb, va, 0.0)
                pltpu.sync_copy(mval_v.at[j], shared_v.at[fidx_v.at[j]],
                                add=True)
                return carry
            lax.fori_loop(0, nrows, row_loop, 0)
            pltpu.sync_copy(shared_v.at[fidx_v.at[nrows - 1]], flush_v)
            plsc.subcore_barrier()
            pltpu.sync_copy(
                shared_v.at[pl.ds(s * TILE_SLICE, TILE_SLICE)],
                wt_hbm.at[pl.ds(gblk * BLK_FLOATS + s * TILE_SLICE,
                                TILE_SLICE)])
            plsc.subcore_barrier()

    return pl.kernel(
        body,
        out_type=jax.ShapeDtypeStruct((D_IN * ROWS,), jnp.float32),
        mesh=plsc.VectorSubcoreMesh(core_axis_name="c", subcore_axis_name="s"),
        scratch_types=[
            pltpu.VMEM((ch_per_tile,), jnp.int32),
            pltpu.VMEM((ch_per_tile,), jnp.int32),
            pltpu.VMEM((ch_per_tile,), jnp.float32),
            pltpu.VMEM((nrows, ROWLEN), jnp.int32),
            pltpu.VMEM((nrows, ROWLEN), jnp.float32),
            pltpu.VMEM((ZCHUNK,), jnp.float32),
            pltpu.VMEM((ROWLEN,), jnp.float32),
            pltpu.VMEM_SHARED((BLK_FLOATS,), jnp.float32),
        ],
    )


def _mm_body(x_ref, w_ref, o_ref):
    o_ref[...] = jnp.maximum(
        jnp.dot(x_ref[...], w_ref[...], preferred_element_type=jnp.float32),
        0.0)


def _matmul_relu(x, wt):
    batch = x.shape[0]
    nbn = 8
    bn = ROWS // nbn
    return pl.pallas_call(
        _mm_body,
        grid=(nbn,),
        in_specs=[
            pl.BlockSpec((batch, D_IN), lambda j: (0, 0)),
            pl.BlockSpec((D_IN, bn), lambda j: (0, j)),
        ],
        out_specs=pl.BlockSpec((batch, bn), lambda j: (0, j)),
        out_shape=jax.ShapeDtypeStruct((batch, ROWS), jnp.float32),
    )(x, wt)


def kernel(input, weight_indices, weight_values):
    child = weight_indices[0].astype(jnp.int32)
    parent = weight_indices[1].astype(jnp.int32)
    vals = weight_values.astype(jnp.float32)
    nnz = vals.shape[0]

    ch_per_tile = -(-nnz // NS)
    ch_per_tile = -(-ch_per_tile // ROWLEN) * ROWLEN
    pad = ch_per_tile * NS - nnz
    child = jnp.concatenate([child, jnp.zeros((pad,), jnp.int32)])
    parent = jnp.concatenate([parent, jnp.zeros((pad,), jnp.int32)])
    vals = jnp.concatenate([vals, jnp.zeros((pad,), jnp.float32)])

    wt_flat = _build_scatter(ch_per_tile)(child, parent, vals)
    wt = wt_flat.reshape(D_IN, ROWS)
    out = _matmul_relu(input, wt)
    return jnp.concatenate([input, out], axis=1)
